# baseline (speedup 1.0000x reference)
import functools

import jax
import jax.numpy as jnp
from jax import lax
from jax.experimental import pallas as pl
from jax.experimental.pallas import tpu as pltpu
from jax.experimental.pallas import tpu_sc as plsc

_BATCH = 16384
_N_NUM = 13
_N_EMB = 26
_EMB_DIM = 16
_N_OH = 13
_OH_CARD = 8

_NC = 2
_NS = 16
_NW = _NC * _NS
_BPW = _BATCH // _NW
_CHUNKS = _BPW // 16

_DROWS = 32
_TAB_SZ = _N_EMB * _DROWS * _EMB_DIM
_W_OFF = _TAB_SZ
_WEMB_OFF = _W_OFF + _N_NUM
_WOH_OFF = _W_OFF + _N_NUM + _N_EMB * _EMB_DIM
_BIAS_OFF = _W_OFF + 533
_TW_SZ = _TAB_SZ + 536


def _full(val):
    return jnp.full((16,), val, jnp.int32)


def _sc_body(num_hbm, idx_hbm, oh_hbm, tw_hbm, out_hbm,
             num_v, idx_v, oh_v, tw_v, d_v, out_v,
             sem_tw, sem_rest):
    wid = lax.axis_index("s") * _NC + lax.axis_index("c")
    base = wid * _BPW

    tw_cp = pltpu.async_copy(tw_hbm, tw_v, sem_tw)
    idx_cp = pltpu.async_copy(idx_hbm.at[:, pl.ds(base, _BPW)], idx_v, sem_rest)
    num_cp = pltpu.async_copy(num_hbm.at[:, pl.ds(base, _BPW)], num_v, sem_rest)
    oh_cp = pltpu.async_copy(oh_hbm.at[:, pl.ds(base, _BPW)], oh_v, sem_rest)
    tw_cp.wait()

    viota = lax.broadcasted_iota(jnp.int32, (16,), 0)

    def d_field(i, carry):
        acc0 = jnp.zeros((16,), jnp.float32)
        acc1 = jnp.zeros((16,), jnp.float32)
        vrow = viota * _EMB_DIM
        for d in range(_EMB_DIM):
            w_sd = plsc.load_gather(tw_v, [_full(_WEMB_OFF + d) + i * _EMB_DIM])
            fbase = i * (_DROWS * _EMB_DIM) + d
            t0 = plsc.load_gather(tw_v, [vrow + fbase])
            t1 = plsc.load_gather(tw_v, [vrow + (fbase + 16 * _EMB_DIM)])
            acc0 = acc0 + t0 * w_sd
            acc1 = acc1 + t1 * w_sd
        d_v[pl.ds(i * _DROWS, 16)] = acc0
        d_v[pl.ds(i * _DROWS + 16, 16)] = acc1
        return carry

    lax.fori_loop(0, _N_EMB, d_field, 0)

    fcb = plsc.load_gather(tw_v, [_full(_BIAS_OFF)])
    wnum = [plsc.load_gather(tw_v, [_full(_W_OFF + j)]) for j in range(_N_NUM)]

    idx_cp.wait()

    def emb_pass(c, carry):
        cb = c * 16
        acc = jnp.zeros((16,), jnp.float32)
        for i in range(_N_EMB):
            iv = idx_v[i, pl.ds(cb, 16)]
            iv = jnp.clip(iv, 0, _N_EMB - 1)
            acc = acc + plsc.load_gather(d_v, [iv + i * _DROWS])
        out_v[pl.ds(cb, 16)] = acc
        return carry

    lax.fori_loop(0, _CHUNKS, emb_pass, 0)

    num_cp.wait()
    oh_cp.wait()

    def tail_pass(c, carry):
        cb = c * 16
        acc = out_v[pl.ds(cb, 16)] + fcb
        for j in range(_N_NUM):
            v = num_v[j, pl.ds(cb, 16)]
            acc = acc + v * wnum[j]
        for j in range(_N_OH):
            ov = oh_v[j, pl.ds(cb, 16)]
            ov = jnp.clip(ov, 0, _OH_CARD - 1)
            acc = acc + plsc.load_gather(tw_v, [ov + (_WOH_OFF + j * _OH_CARD)])
        out_v[pl.ds(cb, 16)] = acc
        return carry

    lax.fori_loop(0, _CHUNKS, tail_pass, 0)

    pltpu.sync_copy(out_v, out_hbm.at[pl.ds(base, _BPW)])


_sc_forward = functools.partial(
    pl.kernel,
    mesh=plsc.VectorSubcoreMesh(core_axis_name="c", subcore_axis_name="s"),
    out_type=jax.ShapeDtypeStruct((_BATCH,), jnp.float32),
    compiler_params=pltpu.CompilerParams(
        needs_layout_passes=False, use_tc_tiling_on_sc=False,
        disable_bounds_checks=True),
    scratch_types=[
        pltpu.VMEM((_N_NUM, _BPW), jnp.float32),
        pltpu.VMEM((_N_EMB, _BPW), jnp.int32),
        pltpu.VMEM((_N_OH, _BPW), jnp.int32),
        pltpu.VMEM((_TW_SZ,), jnp.float32),
        pltpu.VMEM((_N_EMB * _DROWS,), jnp.float32),
        pltpu.VMEM((_BPW,), jnp.float32),
        pltpu.SemaphoreType.DMA,
        pltpu.SemaphoreType.DMA,
    ],
)(_sc_body)


def kernel(num_features, cat_emb_features, cat_one_hot_features, emb_tables, fc_w, fc_b):
    numt = num_features.T
    idxt = cat_emb_features.astype(jnp.int32).T
    oht = cat_one_hot_features.astype(jnp.int32).T
    tw = jnp.concatenate([
        emb_tables[:, :_DROWS, :].reshape(-1),
        fc_w[:, 0],
        fc_b,
        jnp.zeros((_TW_SZ - _TAB_SZ - 534,), jnp.float32),
    ])
    out = _sc_forward(numt, idxt, oht, tw)
    return out.reshape(_BATCH, 1)


# speedup vs baseline: 1.0162x; 1.0162x over previous
import functools

import jax
import jax.numpy as jnp
from jax import lax
from jax.experimental import pallas as pl
from jax.experimental.pallas import tpu as pltpu
from jax.experimental.pallas import tpu_sc as plsc

_BATCH = 16384
_N_NUM = 13
_N_EMB = 26
_EMB_DIM = 16
_N_OH = 13
_OH_CARD = 8

_NC = 2
_NS = 16
_NW = _NC * _NS
_BPW = _BATCH // _NW
_CHUNKS = _BPW // 16

_DROWS = 32
_TAB_SZ = _N_EMB * _DROWS * _EMB_DIM
_W_OFF = _TAB_SZ
_WEMB_OFF = _W_OFF + _N_NUM
_WOH_OFF = _W_OFF + _N_NUM + _N_EMB * _EMB_DIM
_BIAS_OFF = _W_OFF + 533
_TW_SZ = _TAB_SZ + 536


def _full(val):
    return jnp.full((16,), val, jnp.int32)


def _sc_body(num_hbm, idx_hbm, oh_hbm, tw_hbm, out_hbm,
             num_v, idx_v, oh_v, tw_v, d_v, out_v,
             sem_tw, sem_rest):
    wid = lax.axis_index("s") * _NC + lax.axis_index("c")
    base = wid * _BPW

    tw_cp = pltpu.async_copy(tw_hbm, tw_v, sem_tw)
    rest = [
        pltpu.async_copy(num_hbm.at[:, pl.ds(base, _BPW)], num_v, sem_rest),
        pltpu.async_copy(idx_hbm.at[:, pl.ds(base, _BPW)], idx_v, sem_rest),
        pltpu.async_copy(oh_hbm.at[:, pl.ds(base, _BPW)], oh_v, sem_rest),
    ]
    tw_cp.wait()

    viota = lax.broadcasted_iota(jnp.int32, (16,), 0)

    def d_field(i, carry):
        acc0 = jnp.zeros((16,), jnp.float32)
        acc1 = jnp.zeros((16,), jnp.float32)
        vrow = viota * _EMB_DIM
        for d in range(_EMB_DIM):
            w_sd = plsc.load_gather(tw_v, [_full(_WEMB_OFF + d) + i * _EMB_DIM])
            fbase = i * (_DROWS * _EMB_DIM) + d
            t0 = plsc.load_gather(tw_v, [vrow + fbase])
            t1 = plsc.load_gather(tw_v, [vrow + (fbase + 16 * _EMB_DIM)])
            acc0 = acc0 + t0 * w_sd
            acc1 = acc1 + t1 * w_sd
        d_v[pl.ds(i * _DROWS, 16)] = acc0
        d_v[pl.ds(i * _DROWS + 16, 16)] = acc1
        return carry

    lax.fori_loop(0, _N_EMB, d_field, 0)

    fcb = plsc.load_gather(tw_v, [_full(_BIAS_OFF)])
    wnum = [plsc.load_gather(tw_v, [_full(_W_OFF + j)]) for j in range(_N_NUM)]

    for cp in rest:
        cp.wait()

    def chunk(c, carry):
        cb = c * 16
        acc = fcb
        for j in range(_N_NUM):
            v = num_v[j, pl.ds(cb, 16)]
            acc = acc + v * wnum[j]
        for i in range(_N_EMB):
            iv = idx_v[i, pl.ds(cb, 16)]
            iv = jnp.clip(iv, 0, _N_EMB - 1)
            acc = acc + plsc.load_gather(d_v, [iv + i * _DROWS])
        for j in range(_N_OH):
            ov = oh_v[j, pl.ds(cb, 16)]
            ov = jnp.clip(ov, 0, _OH_CARD - 1)
            acc = acc + plsc.load_gather(tw_v, [ov + (_WOH_OFF + j * _OH_CARD)])
        out_v[pl.ds(cb, 16)] = acc
        return carry

    lax.fori_loop(0, _CHUNKS, chunk, 0)

    pltpu.sync_copy(out_v, out_hbm.at[pl.ds(base, _BPW)])


_sc_forward = functools.partial(
    pl.kernel,
    mesh=plsc.VectorSubcoreMesh(core_axis_name="c", subcore_axis_name="s"),
    out_type=jax.ShapeDtypeStruct((_BATCH,), jnp.float32),
    compiler_params=pltpu.CompilerParams(
        needs_layout_passes=False, use_tc_tiling_on_sc=False,
        disable_bounds_checks=True),
    scratch_types=[
        pltpu.VMEM((_N_NUM, _BPW), jnp.float32),
        pltpu.VMEM((_N_EMB, _BPW), jnp.int32),
        pltpu.VMEM((_N_OH, _BPW), jnp.int32),
        pltpu.VMEM((_TW_SZ,), jnp.float32),
        pltpu.VMEM((_N_EMB * _DROWS,), jnp.float32),
        pltpu.VMEM((_BPW,), jnp.float32),
        pltpu.SemaphoreType.DMA,
        pltpu.SemaphoreType.DMA,
    ],
)(_sc_body)


def kernel(num_features, cat_emb_features, cat_one_hot_features, emb_tables, fc_w, fc_b):
    numt = num_features.T
    idxt = cat_emb_features.astype(jnp.int32).T
    oht = cat_one_hot_features.astype(jnp.int32).T
    tw = jnp.concatenate([
        emb_tables[:, :_DROWS, :].reshape(-1),
        fc_w[:, 0],
        fc_b,
        jnp.zeros((_TW_SZ - _TAB_SZ - 534,), jnp.float32),
    ])
    out = _sc_forward(numt, idxt, oht, tw)
    return out.reshape(_BATCH, 1)
